# baseline (speedup 1.0000x reference)
import functools

import jax
import jax.numpy as jnp
from jax import lax
from jax.experimental import pallas as pl
from jax.experimental.pallas import tpu as pltpu
from jax.experimental.pallas import tpu_sc as plsc

_N = 10000
_E = 320000
_C = 32
_NC = 2
_NS = 16
_NW = _NC * _NS
_EPW = _E // _NW
_BLK = 80
_NBLK = _EPW // _BLK
_NPAD = 10240
_ROWS = 3 * _NPAD
_RPS = _ROWS // _NS
_F32 = jnp.float32
_I32 = jnp.int32


def _sc_gat_body(src_hbm, dst_hbm, xl_hbm, xr_hbm, ee_hbm, att_hbm, out_hbm,
                 srcv, dstv, xlv, xrv, eev, stage0, stage1, sidx0, sidx1,
                 attv, eab, obuf, acc, semi, semg, sems0, sems1):
    c = lax.axis_index("c")
    s = lax.axis_index("s")
    ebase = (c * _NS + s) * _EPW

    zero = jnp.zeros((16,), _F32)

    def _zrow(i, _):
        obuf[i, :] = zero
        return 0
    lax.fori_loop(0, 120, _zrow, 0, unroll=8)

    def _zcp(i, _):
        pltpu.sync_copy(obuf, acc.at[pl.ds(s * _RPS + i * 120, 120)])
        return 0
    lax.fori_loop(0, 16, _zcp, 0)

    pltpu.sync_copy(att_hbm, attv)
    plsc.subcore_barrier()

    ii = jnp.arange(16, dtype=_I32)
    lane0 = jnp.where(ii == 0, 1.0, 0.0).astype(_F32)

    def issue_idx(blk, b):
        base = ebase + blk * _BLK
        pltpu.async_copy(src_hbm.at[pl.ds(base, _BLK)], srcv.at[b], semi)
        pltpu.async_copy(dst_hbm.at[pl.ds(base, _BLK)], dstv.at[b], semi)

    def wait_idx(b):
        pltpu.make_async_copy(src_hbm.at[pl.ds(0, _BLK)], srcv.at[b], semi).wait()
        pltpu.make_async_copy(dst_hbm.at[pl.ds(0, _BLK)], dstv.at[b], semi).wait()

    def issue_gath(blk, b):
        base = ebase + blk * _BLK
        pltpu.async_copy(xl_hbm.at[srcv.at[b]], xlv.at[b], semg)
        pltpu.async_copy(xr_hbm.at[dstv.at[b]], xrv.at[b], semg)
        pltpu.async_copy(ee_hbm.at[pl.ds(base, _BLK)], eev.at[b], semg)

    def wait_gath(b):
        pltpu.make_async_copy(xl_hbm.at[srcv.at[b]], xlv.at[b], semg).wait()
        pltpu.make_async_copy(xr_hbm.at[dstv.at[b]], xrv.at[b], semg).wait()
        pltpu.make_async_copy(ee_hbm.at[pl.ds(0, _BLK)], eev.at[b], semg).wait()

    def _ssem(b):
        return sems0 if b == 0 else sems1

    def _stg(b):
        return stage0 if b == 0 else stage1

    def _sdx(b):
        return sidx0 if b == 0 else sidx1

    def compute(b):
        xlb, xrb, eeb, dstb = xlv.at[b], xrv.at[b], eev.at[b], dstv.at[b]
        stg, sdx = _stg(b), _sdx(b)
        for g in range(_BLK // 16):
            eidx = 16 * g + ii

            conts = []
            for cc in range(_C):
                cs = jnp.full((16,), cc, _I32)
                sv = (plsc.load_gather(xlb, [eidx, cs])
                      + plsc.load_gather(xrb, [eidx, cs])
                      + plsc.load_gather(eeb, [eidx, cs]))
                lv = jnp.maximum(sv, 0.2 * sv)
                conts.append(lv * plsc.load_gather(attv, [cs]))
            while len(conts) > 1:
                conts = [a + b for a, b in zip(conts[::2], conts[1::2])]
            eab[:] = jnp.exp(conts[0])

            for l in range(16):
                e = 16 * g + l
                ev = plsc.load_gather(eab, [jnp.full((16,), l, _I32)])
                a0 = xlb[e, pl.ds(0, 16)]
                a1 = xlb[e, pl.ds(16, 16)]
                stg[3 * e, :] = a0 * ev
                stg[3 * e + 1, :] = a1 * ev
                stg[3 * e + 2, :] = ev * lane0

            d3 = 3 * dstb[pl.ds(16 * g, 16)]
            for r in range(3):
                pos = 48 * g + 3 * ii + r
                plsc.store_scatter(sdx, [pos // 120, pos % 120], d3 + r)

    def issue_scat(b):
        sem, stg, sdx = _ssem(b), _stg(b), _sdx(b)
        pltpu.async_copy(stg.at[pl.ds(0, 120)], acc.at[sdx.at[0]], sem, add=True)
        pltpu.async_copy(stg.at[pl.ds(120, 120)], acc.at[sdx.at[1]], sem, add=True)

    def wait_scat(b):
        sem, stg, sdx = _ssem(b), _stg(b), _sdx(b)
        pltpu.make_async_copy(stg.at[pl.ds(0, 120)], acc.at[sdx.at[0]], sem).wait()
        pltpu.make_async_copy(stg.at[pl.ds(120, 120)], acc.at[sdx.at[1]], sem).wait()

    issue_idx(0, 0)
    wait_idx(0)
    issue_gath(0, 0)
    issue_idx(1, 1)

    def outer(j, _):
        for b in (0, 1):
            blk = 2 * j + b
            wait_gath(b)
            wait_idx(1 - b)
            issue_gath(blk + 1, 1 - b)

            @pl.when(j > 0)
            def _():
                wait_scat(b)
            compute(b)
            issue_scat(b)

            @pl.when(blk <= _NBLK - 3)
            def _():
                issue_idx(blk + 2, b)
        return 0
    lax.fori_loop(0, (_NBLK - 1) // 2, outer, 0)
    wait_gath(0)
    wait_scat(0)
    compute(0)
    issue_scat(0)
    wait_scat(1)
    wait_scat(0)

    plsc.subcore_barrier()

    def _ocp(i, _):
        rs = s * _RPS + i * 120
        pltpu.sync_copy(acc.at[pl.ds(rs, 120)], obuf)
        pltpu.sync_copy(obuf, out_hbm.at[c, pl.ds(rs, 120)])
        return 0
    lax.fori_loop(0, 16, _ocp, 0)


_sc_mesh = plsc.VectorSubcoreMesh(
    core_axis_name="c", subcore_axis_name="s", num_cores=_NC, num_subcores=_NS)

_sc_layer = pl.kernel(
    _sc_gat_body,
    out_type=jax.ShapeDtypeStruct((_NC, _ROWS, 16), _F32),
    mesh=_sc_mesh,
    compiler_params=pltpu.CompilerParams(
        needs_layout_passes=False, use_tc_tiling_on_sc=False),
    scratch_types=[
        pltpu.VMEM((2, _BLK), _I32),
        pltpu.VMEM((2, _BLK), _I32),
        pltpu.VMEM((2, _BLK, _C), _F32),
        pltpu.VMEM((2, _BLK, _C), _F32),
        pltpu.VMEM((2, _BLK, _C), _F32),
        pltpu.VMEM((3 * _BLK, 16), _F32),
        pltpu.VMEM((3 * _BLK, 16), _F32),
        pltpu.VMEM((2, 120), _I32),
        pltpu.VMEM((2, 120), _I32),
        pltpu.VMEM((_C,), _F32),
        pltpu.VMEM((16,), _F32),
        pltpu.VMEM((120, 16), _F32),
        pltpu.VMEM_SHARED((_ROWS, 16), _F32),
        pltpu.SemaphoreType.DMA,
        pltpu.SemaphoreType.DMA,
        pltpu.SemaphoreType.DMA,
        pltpu.SemaphoreType.DMA,
    ],
)


def _prep_body(x_ref, wl_ref, wr_ref, bl_ref, br_ref, xl_ref, xr_ref):
    xb = x_ref[...]
    xl_ref[...] = jnp.dot(xb, wl_ref[...], preferred_element_type=_F32) + bl_ref[...]
    xr_ref[...] = jnp.dot(xb, wr_ref[...], preferred_element_type=_F32) + br_ref[...]


def _prep(x, wl, wr, bl, br):
    din = x.shape[1]
    return pl.pallas_call(
        _prep_body,
        grid=(10,),
        in_specs=[pl.BlockSpec((1000, din), lambda i: (i, 0)),
                  pl.BlockSpec((din, _C), lambda i: (0, 0)),
                  pl.BlockSpec((din, _C), lambda i: (0, 0)),
                  pl.BlockSpec((1, _C), lambda i: (0, 0)),
                  pl.BlockSpec((1, _C), lambda i: (0, 0))],
        out_specs=[pl.BlockSpec((1000, _C), lambda i: (i, 0)),
                   pl.BlockSpec((1000, _C), lambda i: (i, 0))],
        out_shape=[jax.ShapeDtypeStruct((_N, _C), _F32)] * 2,
    )(x, wl, wr, bl.reshape(1, -1), br.reshape(1, -1))


def _ee_body(a_ref, w0_ref, w1_ref, w2_ref, o0_ref, o1_ref, o2_ref):
    a = a_ref[...]
    for w_ref, o_ref in ((w0_ref, o0_ref), (w1_ref, o1_ref), (w2_ref, o2_ref)):
        w = w_ref[...]
        o_ref[...] = (a[:, 0:1] * w[0:1, :] + a[:, 1:2] * w[1:2, :]
                      + a[:, 2:3] * w[2:3, :] + a[:, 3:4] * w[3:4, :])


def _ee(edge_attr, we0, we1, we2):
    eb = 8000
    return pl.pallas_call(
        _ee_body,
        grid=(_E // eb,),
        in_specs=[pl.BlockSpec((eb, 4), lambda i: (i, 0))]
                 + [pl.BlockSpec((4, _C), lambda i: (0, 0))] * 3,
        out_specs=[pl.BlockSpec((eb, _C), lambda i: (i, 0))] * 3,
        out_shape=[jax.ShapeDtypeStruct((_E, _C), _F32)] * 3,
    )(edge_attr, we0, we1, we2)


def _combine_body(p_ref, bb_ref, wl_ref, bl_ref, wr_ref, br_ref, xl_ref, xr_ref):
    p = p_ref[0] + p_ref[1]
    den = p[:, 32:33]
    den = jnp.where(den > 0, den, 1.0)
    h = jnp.maximum(p[:, 0:32] / den + bb_ref[...], 0.0)
    xl_ref[...] = jnp.dot(h, wl_ref[...], preferred_element_type=_F32) + bl_ref[...]
    xr_ref[...] = jnp.dot(h, wr_ref[...], preferred_element_type=_F32) + br_ref[...]


def _combine(part, bb, wl, bl, wr, br):
    p = part.reshape(_NC, _NPAD, 48)
    return pl.pallas_call(
        _combine_body,
        grid=(10,),
        in_specs=[pl.BlockSpec((_NC, 1000, 48), lambda i: (0, i, 0)),
                  pl.BlockSpec((1, _C), lambda i: (0, 0)),
                  pl.BlockSpec((_C, _C), lambda i: (0, 0)),
                  pl.BlockSpec((1, _C), lambda i: (0, 0)),
                  pl.BlockSpec((_C, _C), lambda i: (0, 0)),
                  pl.BlockSpec((1, _C), lambda i: (0, 0))],
        out_specs=[pl.BlockSpec((1000, _C), lambda i: (i, 0)),
                   pl.BlockSpec((1000, _C), lambda i: (i, 0))],
        out_shape=[jax.ShapeDtypeStruct((_N, _C), _F32)] * 2,
    )(p, bb.reshape(1, -1), wl, bl.reshape(1, -1), wr, br.reshape(1, -1))


def _final_body(p_ref, bb_ref, w1_ref, b1_ref, w2_ref, b2_ref, out_ref, gacc):
    i = pl.program_id(0)
    p = p_ref[0] + p_ref[1]
    den = p[:, 32:33]
    den = jnp.where(den > 0, den, 1.0)
    h = jnp.maximum(p[:, 0:32] / den + bb_ref[...], 0.0)
    ps = jnp.sum(h, axis=0, keepdims=True)

    @pl.when(i == 0)
    def _():
        gacc[...] = ps

    @pl.when(i > 0)
    def _():
        gacc[...] = gacc[...] + ps

    @pl.when(i == pl.num_programs(0) - 1)
    def _():
        g = gacc[...][0]
        h1 = jnp.sum(w1_ref[...] * g[:, None], axis=0, keepdims=True) + b1_ref[...]
        h1 = jnp.maximum(h1, 0.0)
        o = jnp.sum(w2_ref[...] * h1[0][:, None], axis=0, keepdims=True) + b2_ref[...]
        m = jnp.max(o, axis=-1, keepdims=True)
        e2 = jnp.exp(o - m)
        out_ref[...] = e2 / jnp.sum(e2, axis=-1, keepdims=True)


def _final(part, bb, w1, b1, w2, b2):
    p = part.reshape(_NC, _NPAD, 48)
    return pl.pallas_call(
        _final_body,
        grid=(10,),
        in_specs=[pl.BlockSpec((_NC, 1000, 48), lambda i: (0, i, 0)),
                  pl.BlockSpec((1, _C), lambda i: (0, 0)),
                  pl.BlockSpec((_C, 64), lambda i: (0, 0)),
                  pl.BlockSpec((1, 64), lambda i: (0, 0)),
                  pl.BlockSpec((64, 2), lambda i: (0, 0)),
                  pl.BlockSpec((1, 2), lambda i: (0, 0))],
        out_specs=pl.BlockSpec((1, 2), lambda i: (0, 0)),
        out_shape=jax.ShapeDtypeStruct((1, 2), _F32),
        scratch_shapes=[pltpu.VMEM((1, _C), _F32)],
    )(p, bb.reshape(1, -1), w1, b1.reshape(1, -1), w2, b2.reshape(1, -1))


def kernel(x, edge_index, edge_attr,
           Wl0, Wr0, We0, att0, bl0, br0, b0,
           Wl1, Wr1, We1, att1, bl1, br1, b1,
           Wl2, Wr2, We2, att2, bl2, br2, b2,
           fc1_W, fc1_b, fc2_W, fc2_b):
    src = edge_index[0]
    dst = edge_index[1]
    ee0, ee1, ee2 = _ee(edge_attr, We0, We1, We2)

    xl, xr = _prep(x, Wl0, Wr0, bl0, br0)
    part = _sc_layer(src, dst, xl, xr, ee0, att0)
    xl, xr = _combine(part, b0, Wl1, bl1, Wr1, br1)
    part = _sc_layer(src, dst, xl, xr, ee1, att1)
    xl, xr = _combine(part, b1, Wl2, bl2, Wr2, br2)
    part = _sc_layer(src, dst, xl, xr, ee2, att2)
    out = _final(part, b2, fc1_W, fc1_b, fc2_W, fc2_b)
    return out[0]


# speedup vs baseline: 1.7339x; 1.7339x over previous
import functools

import jax
import jax.numpy as jnp
from jax import lax
from jax.experimental import pallas as pl
from jax.experimental.pallas import tpu as pltpu
from jax.experimental.pallas import tpu_sc as plsc

_N = 10000
_E = 320000
_C = 32
_NC = 2
_NS = 16
_NW = _NC * _NS
_EPW = _E // _NW
_BLK = 80
_NBLK = _EPW // _BLK
_NPAD = 10240
_ROWS = 3 * _NPAD
_RPS = _ROWS // _NS
_F32 = jnp.float32
_I32 = jnp.int32


def _sc_gat_body(src_hbm, dst_hbm, xl_hbm, xr_hbm, ee_hbm, att_hbm, out_hbm,
                 srcv, dstv, xlv, xrv, eev, stage0, stage1, sidx0, sidx1,
                 attv, eab, csbuf, obuf, acc, semi, semg, sems0, sems1):
    c = lax.axis_index("c")
    s = lax.axis_index("s")
    ebase = (c * _NS + s) * _EPW

    zero = jnp.zeros((16,), _F32)

    def _zrow(i, _):
        obuf[i, :] = zero
        return 0
    lax.fori_loop(0, 120, _zrow, 0, unroll=8)

    def _zcp(i, _):
        pltpu.sync_copy(obuf, acc.at[pl.ds(s * _RPS + i * 120, 120)])
        return 0
    lax.fori_loop(0, 16, _zcp, 0)

    pltpu.sync_copy(att_hbm, attv)
    plsc.subcore_barrier()

    ii = jnp.arange(16, dtype=_I32)
    lane0 = jnp.where(ii == 0, 1.0, 0.0).astype(_F32)
    att0 = attv[pl.ds(0, 16)]
    att1 = attv[pl.ds(16, 16)]
    i15 = jnp.full((16,), 15, _I32)

    def issue_idx(blk, b):
        base = ebase + blk * _BLK
        pltpu.async_copy(src_hbm.at[pl.ds(base, _BLK)], srcv.at[b], semi)
        pltpu.async_copy(dst_hbm.at[pl.ds(base, _BLK)], dstv.at[b], semi)

    def wait_idx(b):
        pltpu.make_async_copy(src_hbm.at[pl.ds(0, _BLK)], srcv.at[b], semi).wait()
        pltpu.make_async_copy(dst_hbm.at[pl.ds(0, _BLK)], dstv.at[b], semi).wait()

    def issue_gath(blk, b):
        base = ebase + blk * _BLK
        pltpu.async_copy(xl_hbm.at[srcv.at[b]], xlv.at[b], semg)
        pltpu.async_copy(xr_hbm.at[dstv.at[b]], xrv.at[b], semg)
        pltpu.async_copy(ee_hbm.at[pl.ds(base, _BLK)], eev.at[b], semg)

    def wait_gath(b):
        pltpu.make_async_copy(xl_hbm.at[srcv.at[b]], xlv.at[b], semg).wait()
        pltpu.make_async_copy(xr_hbm.at[dstv.at[b]], xrv.at[b], semg).wait()
        pltpu.make_async_copy(ee_hbm.at[pl.ds(0, _BLK)], eev.at[b], semg).wait()

    def _ssem(b):
        return sems0 if b == 0 else sems1

    def _stg(b):
        return stage0 if b == 0 else stage1

    def _sdx(b):
        return sidx0 if b == 0 else sidx1

    def compute(b):
        xlb, xrb, eeb, dstb = xlv.at[b], xrv.at[b], eev.at[b], dstv.at[b]
        stg, sdx = _stg(b), _sdx(b)
        for g in range(_BLK // 16):
            for l in range(16):
                e = 16 * g + l
                s0 = (xlb[e, pl.ds(0, 16)] + xrb[e, pl.ds(0, 16)]
                      + eeb[e, pl.ds(0, 16)])
                s1 = (xlb[e, pl.ds(16, 16)] + xrb[e, pl.ds(16, 16)]
                      + eeb[e, pl.ds(16, 16)])
                l0 = jnp.maximum(s0, 0.2 * s0)
                l1 = jnp.maximum(s1, 0.2 * s1)
                csbuf[l, :] = plsc.cumsum(l0 * att0 + l1 * att1)
            eab[:] = jnp.exp(plsc.load_gather(csbuf, [ii, i15]))

            for l in range(16):
                e = 16 * g + l
                ev = plsc.load_gather(eab, [jnp.full((16,), l, _I32)])
                a0 = xlb[e, pl.ds(0, 16)]
                a1 = xlb[e, pl.ds(16, 16)]
                stg[3 * e, :] = a0 * ev
                stg[3 * e + 1, :] = a1 * ev
                stg[3 * e + 2, :] = ev * lane0

            d3 = 3 * dstb[pl.ds(16 * g, 16)]
            for r in range(3):
                pos = 48 * g + 3 * ii + r
                plsc.store_scatter(sdx, [pos // 120, pos % 120], d3 + r)

    def issue_scat(b):
        sem, stg, sdx = _ssem(b), _stg(b), _sdx(b)
        pltpu.async_copy(stg.at[pl.ds(0, 120)], acc.at[sdx.at[0]], sem, add=True)
        pltpu.async_copy(stg.at[pl.ds(120, 120)], acc.at[sdx.at[1]], sem, add=True)

    def wait_scat(b):
        sem, stg, sdx = _ssem(b), _stg(b), _sdx(b)
        pltpu.make_async_copy(stg.at[pl.ds(0, 120)], acc.at[sdx.at[0]], sem).wait()
        pltpu.make_async_copy(stg.at[pl.ds(120, 120)], acc.at[sdx.at[1]], sem).wait()

    issue_idx(0, 0)
    wait_idx(0)
    issue_gath(0, 0)
    issue_idx(1, 1)

    def outer(j, _):
        for b in (0, 1):
            blk = 2 * j + b
            wait_gath(b)
            wait_idx(1 - b)
            issue_gath(blk + 1, 1 - b)

            @pl.when(j > 0)
            def _():
                wait_scat(b)
            compute(b)
            issue_scat(b)

            @pl.when(blk <= _NBLK - 3)
            def _():
                issue_idx(blk + 2, b)
        return 0
    lax.fori_loop(0, (_NBLK - 1) // 2, outer, 0)
    wait_gath(0)
    wait_scat(0)
    compute(0)
    issue_scat(0)
    wait_scat(1)
    wait_scat(0)

    plsc.subcore_barrier()

    def _ocp(i, _):
        rs = s * _RPS + i * 120
        pltpu.sync_copy(acc.at[pl.ds(rs, 120)], obuf)
        pltpu.sync_copy(obuf, out_hbm.at[c, pl.ds(rs, 120)])
        return 0
    lax.fori_loop(0, 16, _ocp, 0)


_sc_mesh = plsc.VectorSubcoreMesh(
    core_axis_name="c", subcore_axis_name="s", num_cores=_NC, num_subcores=_NS)

_sc_layer = pl.kernel(
    _sc_gat_body,
    out_type=jax.ShapeDtypeStruct((_NC, _ROWS, 16), _F32),
    mesh=_sc_mesh,
    compiler_params=pltpu.CompilerParams(
        needs_layout_passes=False, use_tc_tiling_on_sc=False),
    scratch_types=[
        pltpu.VMEM((2, _BLK), _I32),
        pltpu.VMEM((2, _BLK), _I32),
        pltpu.VMEM((2, _BLK, _C), _F32),
        pltpu.VMEM((2, _BLK, _C), _F32),
        pltpu.VMEM((2, _BLK, _C), _F32),
        pltpu.VMEM((3 * _BLK, 16), _F32),
        pltpu.VMEM((3 * _BLK, 16), _F32),
        pltpu.VMEM((2, 120), _I32),
        pltpu.VMEM((2, 120), _I32),
        pltpu.VMEM((_C,), _F32),
        pltpu.VMEM((16,), _F32),
        pltpu.VMEM((16, 16), _F32),
        pltpu.VMEM((120, 16), _F32),
        pltpu.VMEM_SHARED((_ROWS, 16), _F32),
        pltpu.SemaphoreType.DMA,
        pltpu.SemaphoreType.DMA,
        pltpu.SemaphoreType.DMA,
        pltpu.SemaphoreType.DMA,
    ],
)


def _prep_body(x_ref, wl_ref, wr_ref, bl_ref, br_ref, xl_ref, xr_ref):
    xb = x_ref[...]
    xl_ref[...] = jnp.dot(xb, wl_ref[...], preferred_element_type=_F32) + bl_ref[...]
    xr_ref[...] = jnp.dot(xb, wr_ref[...], preferred_element_type=_F32) + br_ref[...]


def _prep(x, wl, wr, bl, br):
    din = x.shape[1]
    return pl.pallas_call(
        _prep_body,
        grid=(10,),
        in_specs=[pl.BlockSpec((1000, din), lambda i: (i, 0)),
                  pl.BlockSpec((din, _C), lambda i: (0, 0)),
                  pl.BlockSpec((din, _C), lambda i: (0, 0)),
                  pl.BlockSpec((1, _C), lambda i: (0, 0)),
                  pl.BlockSpec((1, _C), lambda i: (0, 0))],
        out_specs=[pl.BlockSpec((1000, _C), lambda i: (i, 0)),
                   pl.BlockSpec((1000, _C), lambda i: (i, 0))],
        out_shape=[jax.ShapeDtypeStruct((_N, _C), _F32)] * 2,
    )(x, wl, wr, bl.reshape(1, -1), br.reshape(1, -1))


def _ee_body(a_ref, w0_ref, w1_ref, w2_ref, o0_ref, o1_ref, o2_ref):
    a = a_ref[...]
    for w_ref, o_ref in ((w0_ref, o0_ref), (w1_ref, o1_ref), (w2_ref, o2_ref)):
        w = w_ref[...]
        o_ref[...] = (a[:, 0:1] * w[0:1, :] + a[:, 1:2] * w[1:2, :]
                      + a[:, 2:3] * w[2:3, :] + a[:, 3:4] * w[3:4, :])


def _ee(edge_attr, we0, we1, we2):
    eb = 8000
    return pl.pallas_call(
        _ee_body,
        grid=(_E // eb,),
        in_specs=[pl.BlockSpec((eb, 4), lambda i: (i, 0))]
                 + [pl.BlockSpec((4, _C), lambda i: (0, 0))] * 3,
        out_specs=[pl.BlockSpec((eb, _C), lambda i: (i, 0))] * 3,
        out_shape=[jax.ShapeDtypeStruct((_E, _C), _F32)] * 3,
    )(edge_attr, we0, we1, we2)


def _combine_body(p_ref, bb_ref, wl_ref, bl_ref, wr_ref, br_ref, xl_ref, xr_ref):
    p = p_ref[0] + p_ref[1]
    den = p[:, 32:33]
    den = jnp.where(den > 0, den, 1.0)
    h = jnp.maximum(p[:, 0:32] / den + bb_ref[...], 0.0)
    xl_ref[...] = jnp.dot(h, wl_ref[...], preferred_element_type=_F32) + bl_ref[...]
    xr_ref[...] = jnp.dot(h, wr_ref[...], preferred_element_type=_F32) + br_ref[...]


def _combine(part, bb, wl, bl, wr, br):
    p = part.reshape(_NC, _NPAD, 48)
    return pl.pallas_call(
        _combine_body,
        grid=(10,),
        in_specs=[pl.BlockSpec((_NC, 1000, 48), lambda i: (0, i, 0)),
                  pl.BlockSpec((1, _C), lambda i: (0, 0)),
                  pl.BlockSpec((_C, _C), lambda i: (0, 0)),
                  pl.BlockSpec((1, _C), lambda i: (0, 0)),
                  pl.BlockSpec((_C, _C), lambda i: (0, 0)),
                  pl.BlockSpec((1, _C), lambda i: (0, 0))],
        out_specs=[pl.BlockSpec((1000, _C), lambda i: (i, 0)),
                   pl.BlockSpec((1000, _C), lambda i: (i, 0))],
        out_shape=[jax.ShapeDtypeStruct((_N, _C), _F32)] * 2,
    )(p, bb.reshape(1, -1), wl, bl.reshape(1, -1), wr, br.reshape(1, -1))


def _final_body(p_ref, bb_ref, w1_ref, b1_ref, w2_ref, b2_ref, out_ref, gacc):
    i = pl.program_id(0)
    p = p_ref[0] + p_ref[1]
    den = p[:, 32:33]
    den = jnp.where(den > 0, den, 1.0)
    h = jnp.maximum(p[:, 0:32] / den + bb_ref[...], 0.0)
    ps = jnp.sum(h, axis=0, keepdims=True)

    @pl.when(i == 0)
    def _():
        gacc[...] = ps

    @pl.when(i > 0)
    def _():
        gacc[...] = gacc[...] + ps

    @pl.when(i == pl.num_programs(0) - 1)
    def _():
        g = gacc[...][0]
        h1 = jnp.sum(w1_ref[...] * g[:, None], axis=0, keepdims=True) + b1_ref[...]
        h1 = jnp.maximum(h1, 0.0)
        o = jnp.sum(w2_ref[...] * h1[0][:, None], axis=0, keepdims=True) + b2_ref[...]
        m = jnp.max(o, axis=-1, keepdims=True)
        e2 = jnp.exp(o - m)
        out_ref[...] = e2 / jnp.sum(e2, axis=-1, keepdims=True)


def _final(part, bb, w1, b1, w2, b2):
    p = part.reshape(_NC, _NPAD, 48)
    return pl.pallas_call(
        _final_body,
        grid=(10,),
        in_specs=[pl.BlockSpec((_NC, 1000, 48), lambda i: (0, i, 0)),
                  pl.BlockSpec((1, _C), lambda i: (0, 0)),
                  pl.BlockSpec((_C, 64), lambda i: (0, 0)),
                  pl.BlockSpec((1, 64), lambda i: (0, 0)),
                  pl.BlockSpec((64, 2), lambda i: (0, 0)),
                  pl.BlockSpec((1, 2), lambda i: (0, 0))],
        out_specs=pl.BlockSpec((1, 2), lambda i: (0, 0)),
        out_shape=jax.ShapeDtypeStruct((1, 2), _F32),
        scratch_shapes=[pltpu.VMEM((1, _C), _F32)],
    )(p, bb.reshape(1, -1), w1, b1.reshape(1, -1), w2, b2.reshape(1, -1))


def kernel(x, edge_index, edge_attr,
           Wl0, Wr0, We0, att0, bl0, br0, b0,
           Wl1, Wr1, We1, att1, bl1, br1, b1,
           Wl2, Wr2, We2, att2, bl2, br2, b2,
           fc1_W, fc1_b, fc2_W, fc2_b):
    src = edge_index[0]
    dst = edge_index[1]
    ee0, ee1, ee2 = _ee(edge_attr, We0, We1, We2)

    xl, xr = _prep(x, Wl0, Wr0, bl0, br0)
    part = _sc_layer(src, dst, xl, xr, ee0, att0)
    xl, xr = _combine(part, b0, Wl1, bl1, Wr1, br1)
    part = _sc_layer(src, dst, xl, xr, ee1, att1)
    xl, xr = _combine(part, b1, Wl2, bl2, Wr2, br2)
    part = _sc_layer(src, dst, xl, xr, ee2, att2)
    out = _final(part, b2, fc1_W, fc1_b, fc2_W, fc2_b)
    return out[0]


# speedup vs baseline: 1.9877x; 1.1464x over previous
import functools

import jax
import jax.numpy as jnp
from jax import lax
from jax.experimental import pallas as pl
from jax.experimental.pallas import tpu as pltpu
from jax.experimental.pallas import tpu_sc as plsc

_N = 10000
_E = 320000
_C = 32
_NC = 2
_NS = 16
_NW = _NC * _NS
_EPW = _E // _NW
_BLK = 80
_NBLK = _EPW // _BLK
_NPAD = 10240
_ROWS = 3 * _NPAD
_RPS = _ROWS // _NS
_F32 = jnp.float32
_I32 = jnp.int32


def _sc_gat_body(src_hbm, dst_hbm, xl_hbm, xr_hbm, ee_hbm, att_hbm, out_hbm,
                 srcv, dstv, xlv, xrv, eev, stage0, stage1, sidx0, sidx1,
                 attv, eab, csbuf, obuf, acc, semi, semg, sems0, sems1):
    c = lax.axis_index("c")
    s = lax.axis_index("s")
    ebase = (c * _NS + s) * _EPW

    zero = jnp.zeros((16,), _F32)

    def _zrow(i, _):
        obuf[i, :] = zero
        return 0
    lax.fori_loop(0, 120, _zrow, 0, unroll=8)

    def _zcp(i, _):
        pltpu.sync_copy(obuf, acc.at[pl.ds(s * _RPS + i * 120, 120)])
        return 0
    lax.fori_loop(0, 16, _zcp, 0)

    pltpu.sync_copy(att_hbm, attv)
    plsc.subcore_barrier()

    ii = jnp.arange(16, dtype=_I32)
    lane0 = jnp.where(ii == 0, 1.0, 0.0).astype(_F32)
    att0 = attv[pl.ds(0, 16)]
    att1 = attv[pl.ds(16, 16)]
    i15 = jnp.full((16,), 15, _I32)

    def _splat_lane(v, l):
        return lax.gather(
            v, jnp.full((16, 1), l, _I32),
            dimension_numbers=lax.GatherDimensionNumbers(
                offset_dims=(), collapsed_slice_dims=(0,),
                start_index_map=(0,)),
            slice_sizes=(1,), mode=lax.GatherScatterMode.PROMISE_IN_BOUNDS)

    def issue_idx(blk, b):
        base = ebase + blk * _BLK
        pltpu.async_copy(src_hbm.at[pl.ds(base, _BLK)], srcv.at[b], semi)
        pltpu.async_copy(dst_hbm.at[pl.ds(base, _BLK)], dstv.at[b], semi)

    def wait_idx(b):
        pltpu.make_async_copy(src_hbm.at[pl.ds(0, _BLK)], srcv.at[b], semi).wait()
        pltpu.make_async_copy(dst_hbm.at[pl.ds(0, _BLK)], dstv.at[b], semi).wait()

    def issue_gath(blk, b):
        base = ebase + blk * _BLK
        pltpu.async_copy(xl_hbm.at[srcv.at[b]], xlv.at[b], semg)
        pltpu.async_copy(xr_hbm.at[dstv.at[b]], xrv.at[b], semg)
        pltpu.async_copy(ee_hbm.at[pl.ds(base, _BLK)], eev.at[b], semg)

    def wait_gath(b):
        pltpu.make_async_copy(xl_hbm.at[srcv.at[b]], xlv.at[b], semg).wait()
        pltpu.make_async_copy(xr_hbm.at[dstv.at[b]], xrv.at[b], semg).wait()
        pltpu.make_async_copy(ee_hbm.at[pl.ds(0, _BLK)], eev.at[b], semg).wait()

    def _ssem(b):
        return sems0 if b == 0 else sems1

    def _stg(b):
        return stage0 if b == 0 else stage1

    def _sdx(b):
        return sidx0 if b == 0 else sidx1

    def compute(b):
        xlb, xrb, eeb, dstb = xlv.at[b], xrv.at[b], eev.at[b], dstv.at[b]
        stg, sdx = _stg(b), _sdx(b)
        for g in range(_BLK // 16):
            for l in range(16):
                e = 16 * g + l
                s0 = (xlb[e, pl.ds(0, 16)] + xrb[e, pl.ds(0, 16)]
                      + eeb[e, pl.ds(0, 16)])
                s1 = (xlb[e, pl.ds(16, 16)] + xrb[e, pl.ds(16, 16)]
                      + eeb[e, pl.ds(16, 16)])
                l0 = jnp.maximum(s0, 0.2 * s0)
                l1 = jnp.maximum(s1, 0.2 * s1)
                csbuf[l, :] = plsc.cumsum(l0 * att0 + l1 * att1)
            ea16 = jnp.exp(plsc.load_gather(csbuf, [ii, i15]))

            for l in range(16):
                e = 16 * g + l
                ev = _splat_lane(ea16, l)
                a0 = xlb[e, pl.ds(0, 16)]
                a1 = xlb[e, pl.ds(16, 16)]
                stg[3 * e, :] = a0 * ev
                stg[3 * e + 1, :] = a1 * ev
                stg[3 * e + 2, :] = ev * lane0

            d3 = 3 * dstb[pl.ds(16 * g, 16)]
            for r in range(3):
                pos = 48 * g + 3 * ii + r
                plsc.store_scatter(sdx, [pos // 120, pos % 120], d3 + r)

    def issue_scat(b):
        sem, stg, sdx = _ssem(b), _stg(b), _sdx(b)
        pltpu.async_copy(stg.at[pl.ds(0, 120)], acc.at[sdx.at[0]], sem, add=True)
        pltpu.async_copy(stg.at[pl.ds(120, 120)], acc.at[sdx.at[1]], sem, add=True)

    def wait_scat(b):
        sem, stg, sdx = _ssem(b), _stg(b), _sdx(b)
        pltpu.make_async_copy(stg.at[pl.ds(0, 120)], acc.at[sdx.at[0]], sem).wait()
        pltpu.make_async_copy(stg.at[pl.ds(120, 120)], acc.at[sdx.at[1]], sem).wait()

    issue_idx(0, 0)
    wait_idx(0)
    issue_gath(0, 0)
    issue_idx(1, 1)

    def outer(j, _):
        for b in (0, 1):
            blk = 2 * j + b
            wait_gath(b)
            wait_idx(1 - b)
            issue_gath(blk + 1, 1 - b)

            @pl.when(j > 0)
            def _():
                wait_scat(b)
            compute(b)
            issue_scat(b)

            @pl.when(blk <= _NBLK - 3)
            def _():
                issue_idx(blk + 2, b)
        return 0
    lax.fori_loop(0, (_NBLK - 1) // 2, outer, 0)
    wait_gath(0)
    wait_scat(0)
    compute(0)
    issue_scat(0)
    wait_scat(1)
    wait_scat(0)

    plsc.subcore_barrier()

    def _ocp(i, _):
        rs = s * _RPS + i * 120
        pltpu.sync_copy(acc.at[pl.ds(rs, 120)], obuf)
        pltpu.sync_copy(obuf, out_hbm.at[c, pl.ds(rs, 120)])
        return 0
    lax.fori_loop(0, 16, _ocp, 0)


_sc_mesh = plsc.VectorSubcoreMesh(
    core_axis_name="c", subcore_axis_name="s", num_cores=_NC, num_subcores=_NS)

_sc_layer = pl.kernel(
    _sc_gat_body,
    out_type=jax.ShapeDtypeStruct((_NC, _ROWS, 16), _F32),
    mesh=_sc_mesh,
    compiler_params=pltpu.CompilerParams(
        needs_layout_passes=False, use_tc_tiling_on_sc=False),
    scratch_types=[
        pltpu.VMEM((2, _BLK), _I32),
        pltpu.VMEM((2, _BLK), _I32),
        pltpu.VMEM((2, _BLK, _C), _F32),
        pltpu.VMEM((2, _BLK, _C), _F32),
        pltpu.VMEM((2, _BLK, _C), _F32),
        pltpu.VMEM((3 * _BLK, 16), _F32),
        pltpu.VMEM((3 * _BLK, 16), _F32),
        pltpu.VMEM((2, 120), _I32),
        pltpu.VMEM((2, 120), _I32),
        pltpu.VMEM((_C,), _F32),
        pltpu.VMEM((16,), _F32),
        pltpu.VMEM((16, 16), _F32),
        pltpu.VMEM((120, 16), _F32),
        pltpu.VMEM_SHARED((_ROWS, 16), _F32),
        pltpu.SemaphoreType.DMA,
        pltpu.SemaphoreType.DMA,
        pltpu.SemaphoreType.DMA,
        pltpu.SemaphoreType.DMA,
    ],
)


def _prep_body(x_ref, wl_ref, wr_ref, bl_ref, br_ref, xl_ref, xr_ref):
    xb = x_ref[...]
    xl_ref[...] = jnp.dot(xb, wl_ref[...], preferred_element_type=_F32) + bl_ref[...]
    xr_ref[...] = jnp.dot(xb, wr_ref[...], preferred_element_type=_F32) + br_ref[...]


def _prep(x, wl, wr, bl, br):
    din = x.shape[1]
    return pl.pallas_call(
        _prep_body,
        grid=(10,),
        in_specs=[pl.BlockSpec((1000, din), lambda i: (i, 0)),
                  pl.BlockSpec((din, _C), lambda i: (0, 0)),
                  pl.BlockSpec((din, _C), lambda i: (0, 0)),
                  pl.BlockSpec((1, _C), lambda i: (0, 0)),
                  pl.BlockSpec((1, _C), lambda i: (0, 0))],
        out_specs=[pl.BlockSpec((1000, _C), lambda i: (i, 0)),
                   pl.BlockSpec((1000, _C), lambda i: (i, 0))],
        out_shape=[jax.ShapeDtypeStruct((_N, _C), _F32)] * 2,
    )(x, wl, wr, bl.reshape(1, -1), br.reshape(1, -1))


def _ee_body(a_ref, w0_ref, w1_ref, w2_ref, o0_ref, o1_ref, o2_ref):
    a = a_ref[...]
    for w_ref, o_ref in ((w0_ref, o0_ref), (w1_ref, o1_ref), (w2_ref, o2_ref)):
        w = w_ref[...]
        o_ref[...] = (a[:, 0:1] * w[0:1, :] + a[:, 1:2] * w[1:2, :]
                      + a[:, 2:3] * w[2:3, :] + a[:, 3:4] * w[3:4, :])


def _ee(edge_attr, we0, we1, we2):
    eb = 8000
    return pl.pallas_call(
        _ee_body,
        grid=(_E // eb,),
        in_specs=[pl.BlockSpec((eb, 4), lambda i: (i, 0))]
                 + [pl.BlockSpec((4, _C), lambda i: (0, 0))] * 3,
        out_specs=[pl.BlockSpec((eb, _C), lambda i: (i, 0))] * 3,
        out_shape=[jax.ShapeDtypeStruct((_E, _C), _F32)] * 3,
    )(edge_attr, we0, we1, we2)


def _combine_body(p_ref, bb_ref, wl_ref, bl_ref, wr_ref, br_ref, xl_ref, xr_ref):
    p = p_ref[0] + p_ref[1]
    den = p[:, 32:33]
    den = jnp.where(den > 0, den, 1.0)
    h = jnp.maximum(p[:, 0:32] / den + bb_ref[...], 0.0)
    xl_ref[...] = jnp.dot(h, wl_ref[...], preferred_element_type=_F32) + bl_ref[...]
    xr_ref[...] = jnp.dot(h, wr_ref[...], preferred_element_type=_F32) + br_ref[...]


def _combine(part, bb, wl, bl, wr, br):
    p = part.reshape(_NC, _NPAD, 48)
    return pl.pallas_call(
        _combine_body,
        grid=(10,),
        in_specs=[pl.BlockSpec((_NC, 1000, 48), lambda i: (0, i, 0)),
                  pl.BlockSpec((1, _C), lambda i: (0, 0)),
                  pl.BlockSpec((_C, _C), lambda i: (0, 0)),
                  pl.BlockSpec((1, _C), lambda i: (0, 0)),
                  pl.BlockSpec((_C, _C), lambda i: (0, 0)),
                  pl.BlockSpec((1, _C), lambda i: (0, 0))],
        out_specs=[pl.BlockSpec((1000, _C), lambda i: (i, 0)),
                   pl.BlockSpec((1000, _C), lambda i: (i, 0))],
        out_shape=[jax.ShapeDtypeStruct((_N, _C), _F32)] * 2,
    )(p, bb.reshape(1, -1), wl, bl.reshape(1, -1), wr, br.reshape(1, -1))


def _final_body(p_ref, bb_ref, w1_ref, b1_ref, w2_ref, b2_ref, out_ref, gacc):
    i = pl.program_id(0)
    p = p_ref[0] + p_ref[1]
    den = p[:, 32:33]
    den = jnp.where(den > 0, den, 1.0)
    h = jnp.maximum(p[:, 0:32] / den + bb_ref[...], 0.0)
    ps = jnp.sum(h, axis=0, keepdims=True)

    @pl.when(i == 0)
    def _():
        gacc[...] = ps

    @pl.when(i > 0)
    def _():
        gacc[...] = gacc[...] + ps

    @pl.when(i == pl.num_programs(0) - 1)
    def _():
        g = gacc[...][0]
        h1 = jnp.sum(w1_ref[...] * g[:, None], axis=0, keepdims=True) + b1_ref[...]
        h1 = jnp.maximum(h1, 0.0)
        o = jnp.sum(w2_ref[...] * h1[0][:, None], axis=0, keepdims=True) + b2_ref[...]
        m = jnp.max(o, axis=-1, keepdims=True)
        e2 = jnp.exp(o - m)
        out_ref[...] = e2 / jnp.sum(e2, axis=-1, keepdims=True)


def _final(part, bb, w1, b1, w2, b2):
    p = part.reshape(_NC, _NPAD, 48)
    return pl.pallas_call(
        _final_body,
        grid=(10,),
        in_specs=[pl.BlockSpec((_NC, 1000, 48), lambda i: (0, i, 0)),
                  pl.BlockSpec((1, _C), lambda i: (0, 0)),
                  pl.BlockSpec((_C, 64), lambda i: (0, 0)),
                  pl.BlockSpec((1, 64), lambda i: (0, 0)),
                  pl.BlockSpec((64, 2), lambda i: (0, 0)),
                  pl.BlockSpec((1, 2), lambda i: (0, 0))],
        out_specs=pl.BlockSpec((1, 2), lambda i: (0, 0)),
        out_shape=jax.ShapeDtypeStruct((1, 2), _F32),
        scratch_shapes=[pltpu.VMEM((1, _C), _F32)],
    )(p, bb.reshape(1, -1), w1, b1.reshape(1, -1), w2, b2.reshape(1, -1))


def kernel(x, edge_index, edge_attr,
           Wl0, Wr0, We0, att0, bl0, br0, b0,
           Wl1, Wr1, We1, att1, bl1, br1, b1,
           Wl2, Wr2, We2, att2, bl2, br2, b2,
           fc1_W, fc1_b, fc2_W, fc2_b):
    src = edge_index[0]
    dst = edge_index[1]
    ee0, ee1, ee2 = _ee(edge_attr, We0, We1, We2)

    xl, xr = _prep(x, Wl0, Wr0, bl0, br0)
    part = _sc_layer(src, dst, xl, xr, ee0, att0)
    xl, xr = _combine(part, b0, Wl1, bl1, Wr1, br1)
    part = _sc_layer(src, dst, xl, xr, ee1, att1)
    xl, xr = _combine(part, b1, Wl2, bl2, Wr2, br2)
    part = _sc_layer(src, dst, xl, xr, ee2, att2)
    out = _final(part, b2, fc1_W, fc1_b, fc2_W, fc2_b)
    return out[0]
